# baseline (speedup 1.0000x reference)
import functools

import jax
import jax.numpy as jnp
from jax import lax
from jax.experimental import pallas as pl
from jax.experimental.pallas import tpu as pltpu
from jax.experimental.pallas import tpu_sc as plsc

N = 10000
D = 128
E = 320000
ALPHA = 0.1

NC = 2
NS = 16
CHUNK = 128
CPW = 80
E_PAD = NC * NS * CHUNK * CPW
NP = 10240
ROWS_PER_SUB = NP // NS
ZROWS = 64


def _segsum_kernel(table, sidx, didx, vals, out,
                   zbuf, rows0, rows1,
                   cidx0, cidx1, cdidx0, cdidx1, cvals0, cvals1,
                   scidx0, scidx1,
                   sem_g0, sem_g1, sem_s0, sem_s1, sem_i0, sem_i1, acc):
    c = lax.axis_index("c")
    s = lax.axis_index("s")
    wid = s * NC + c
    rows = (rows0, rows1)
    cidx = (cidx0, cidx1)
    cdidx = (cdidx0, cdidx1)
    cvals = (cvals0, cvals1)
    scidx = (scidx0, scidx1)
    sem_g = (sem_g0, sem_g1)
    sem_s = (sem_s0, sem_s1)
    sem_i = (sem_i0, sem_i1)

    with jax.named_scope("acc_zero"):
        def _zrow(i, _):
            for j in range(D // 16):
                zbuf[i, pl.ds(j * 16, 16)] = jnp.zeros((16,), jnp.float32)
            return 0
        lax.fori_loop(0, ZROWS, _zrow, 0)
        for r in range(ROWS_PER_SUB // ZROWS):
            pltpu.sync_copy(zbuf, acc.at[pl.ds(s * ROWS_PER_SUB + r * ZROWS, ZROWS)])
        plsc.subcore_barrier()

    def _base(t):
        return (wid * CPW + t) * CHUNK

    def _prefetch_idx(t, b):
        pltpu.async_copy(sidx.at[pl.ds(_base(t), CHUNK)], cidx[b], sem_i[b])
        pltpu.async_copy(didx.at[pl.ds(_base(t), CHUNK)], cdidx[b], sem_i[b])
        pltpu.async_copy(vals.at[pl.ds(_base(t), CHUNK)], cvals[b], sem_i[b])

    def _idx_wait(b):
        pltpu.make_async_copy(sidx.at[pl.ds(0, CHUNK)], cidx[b], sem_i[b]).wait()
        pltpu.make_async_copy(didx.at[pl.ds(0, CHUNK)], cdidx[b], sem_i[b]).wait()
        pltpu.make_async_copy(vals.at[pl.ds(0, CHUNK)], cvals[b], sem_i[b]).wait()

    def _gather_issue(b):
        pltpu.async_copy(table.at[cidx[b]], rows[b], sem_g[b])

    def _gather_wait(b):
        pltpu.make_async_copy(table.at[pl.ds(0, CHUNK)], rows[b], sem_g[b]).wait()

    def _scatter_issue(b):
        for j in range(CHUNK // 16):
            sl = pl.ds(j * 16, 16)
            scidx[b][sl] = cdidx[b][sl]
        pltpu.async_copy(rows[b], acc.at[scidx[b]], sem_s[b], add=True)

    def _scatter_wait(b):
        pltpu.make_async_copy(table.at[pl.ds(0, CHUNK)], rows[b], sem_s[b]).wait()

    def _scale(b):
        def _g(g, _):
            vv = cvals[b][pl.ds(g * 16, 16)]
            for k in range(16):
                sv = lax.broadcast(vv[k], (16,))
                row = g * 16 + k
                for j in range(D // 16):
                    sl = pl.ds(j * 16, 16)
                    rows[b][row, sl] = rows[b][row, sl] * sv
            return 0
        lax.fori_loop(0, CHUNK // 16, _g, 0)

    def _chunk(t, b, wait_prev_scatter, issue_next, prefetch2):
        _gather_wait(b)
        if wait_prev_scatter:
            _scatter_wait(1 - b)
        if issue_next:
            _idx_wait(1 - b)
            _gather_issue(1 - b)
        _scale(b)
        _scatter_issue(b)
        if prefetch2:
            _prefetch_idx(t + 2, b)

    with jax.named_scope("edge_pipe"):
        pltpu.sync_copy(sidx.at[pl.ds(_base(0), CHUNK)], cidx[0])
        pltpu.sync_copy(didx.at[pl.ds(_base(0), CHUNK)], cdidx[0])
        pltpu.sync_copy(vals.at[pl.ds(_base(0), CHUNK)], cvals[0])
        _gather_issue(0)
        _prefetch_idx(1, 1)
        _chunk(0, 0, False, True, True)
        _chunk(1, 1, True, True, True)

        def _pair(i, _):
            _chunk(2 * i, 0, True, True, True)
            _chunk(2 * i + 1, 1, True, True, True)
            return 0
        lax.fori_loop(1, CPW // 2 - 1, _pair, 0)

        _chunk(CPW - 2, 0, True, True, False)
        _chunk(CPW - 1, 1, True, False, False)
        _scatter_wait(1)
        plsc.subcore_barrier()

    with jax.named_scope("acc_flush"):
        pltpu.sync_copy(acc.at[pl.ds(s * ROWS_PER_SUB, ROWS_PER_SUB)],
                        out.at[c, pl.ds(s * ROWS_PER_SUB, ROWS_PER_SUB)])


@jax.jit
def _segsum(table, sidx, didx, vals):
    mesh = plsc.VectorSubcoreMesh(core_axis_name="c", subcore_axis_name="s")
    f = functools.partial(
        pl.kernel,
        mesh=mesh,
        out_type=jax.ShapeDtypeStruct((NC, NP, D), jnp.float32),
        scratch_types=[
            pltpu.VMEM((ZROWS, D), jnp.float32),
            pltpu.VMEM((CHUNK, D), jnp.float32),
            pltpu.VMEM((CHUNK, D), jnp.float32),
            pltpu.VMEM((CHUNK,), jnp.int32),
            pltpu.VMEM((CHUNK,), jnp.int32),
            pltpu.VMEM((CHUNK,), jnp.int32),
            pltpu.VMEM((CHUNK,), jnp.int32),
            pltpu.VMEM((CHUNK,), jnp.float32),
            pltpu.VMEM((CHUNK,), jnp.float32),
            pltpu.VMEM((CHUNK,), jnp.int32),
            pltpu.VMEM((CHUNK,), jnp.int32),
            pltpu.SemaphoreType.DMA,
            pltpu.SemaphoreType.DMA,
            pltpu.SemaphoreType.DMA,
            pltpu.SemaphoreType.DMA,
            pltpu.SemaphoreType.DMA,
            pltpu.SemaphoreType.DMA,
            pltpu.VMEM_SHARED((NP, D), jnp.float32),
        ],
    )(_segsum_kernel)
    return f(table, sidx, didx, vals)[:, :N]


def _ho_body(p_ref, w_ref, b_ref, o_ref):
    x = p_ref[0] + p_ref[1]
    y = jnp.dot(x, w_ref[...], preferred_element_type=jnp.float32) + b_ref[...]
    o_ref[...] = jnp.where(y >= 0, y, ALPHA * y)


@jax.jit
def _ho(partials, w, b):
    blk = 1000
    grid = N // blk
    return pl.pallas_call(
        _ho_body,
        grid=(grid,),
        in_specs=[
            pl.BlockSpec((NC, blk, D), lambda i: (0, i, 0)),
            pl.BlockSpec((D, D), lambda i: (0, 0)),
            pl.BlockSpec((1, D), lambda i: (0, 0)),
        ],
        out_specs=pl.BlockSpec((blk, D), lambda i: (i, 0)),
        out_shape=jax.ShapeDtypeStruct((N, D), jnp.float32),
    )(partials, w, b.reshape(1, D))


def _final_body(p_ref, fea_ref, wt_ref, wb_ref, b_ref, o_ref):
    x = p_ref[0] + p_ref[1]
    x = jnp.where(x >= 0, x, ALPHA * x)
    y = (jnp.dot(x, wt_ref[...], preferred_element_type=jnp.float32)
         + jnp.dot(fea_ref[...], wb_ref[...], preferred_element_type=jnp.float32)
         + b_ref[...])
    o_ref[...] = jnp.maximum(y, 0.0)


@jax.jit
def _final(partials, fea, w_top, w_bot, b):
    blk = 1000
    grid = N // blk
    return pl.pallas_call(
        _final_body,
        grid=(grid,),
        in_specs=[
            pl.BlockSpec((NC, blk, D), lambda i: (0, i, 0)),
            pl.BlockSpec((blk, D), lambda i: (i, 0)),
            pl.BlockSpec((D, D), lambda i: (0, 0)),
            pl.BlockSpec((D, D), lambda i: (0, 0)),
            pl.BlockSpec((1, D), lambda i: (0, 0)),
        ],
        out_specs=pl.BlockSpec((blk, D), lambda i: (i, 0)),
        out_shape=jax.ShapeDtypeStruct((N, D), jnp.float32),
    )(partials, fea, w_top, w_bot, b.reshape(1, D))


def kernel(ufea, vfea, edge_index, uv_vals, vu_vals, gc1_W, gc1_b, gc2_W,
           gc2_b, user_union_W, user_union_b, item_union_W, item_union_b):
    u_idx = edge_index[0].astype(jnp.int32)
    v_idx = edge_index[1].astype(jnp.int32)
    pad = E_PAD - E
    u_pad = jnp.concatenate([u_idx, jnp.zeros((pad,), jnp.int32)])
    v_pad = jnp.concatenate([v_idx, jnp.zeros((pad,), jnp.int32)])
    uv_pad = jnp.concatenate([uv_vals, jnp.zeros((pad,), jnp.float32)])
    vu_pad = jnp.concatenate([vu_vals, jnp.zeros((pad,), jnp.float32)])

    s1 = _segsum(ufea, u_pad, v_pad, vu_pad)
    s2 = _segsum(vfea, v_pad, u_pad, uv_pad)
    user_ho = _ho(s1, gc1_W, gc1_b)
    item_ho = _ho(s2, gc2_W, gc2_b)

    s3 = _segsum(user_ho, v_pad, u_pad, uv_pad)
    s4 = _segsum(item_ho, u_pad, v_pad, vu_pad)

    user = _final(s3, ufea, user_union_W[:D], user_union_W[D:], user_union_b)
    item = _final(s4, vfea, item_union_W[:D], item_union_W[D:], item_union_b)
    return (user, item)


# speedup vs baseline: 1.0891x; 1.0891x over previous
import functools

import jax
import jax.numpy as jnp
from jax import lax
from jax.experimental import pallas as pl
from jax.experimental.pallas import tpu as pltpu
from jax.experimental.pallas import tpu_sc as plsc

N = 10000
D = 128
E = 320000
ALPHA = 0.1

NC = 2
NS = 16
CHUNK = 128
CPW0 = 122
CPW1 = 38
E_PAD = NS * CHUNK * (CPW0 + CPW1)
NP = 10240
ROWS_PER_SUB = NP // NS
ZROWS = 64


def _segsum_kernel(table, sidx, didx, vals, out,
                   zbuf, rows0, rows1,
                   cidx0, cidx1, cdidx0, cdidx1, cvals0, cvals1,
                   scidx0, scidx1,
                   sem_g0, sem_g1, sem_s0, sem_s1, sem_i0, sem_i1, acc):
    c = lax.axis_index("c")
    s = lax.axis_index("s")
    rows = (rows0, rows1)
    cidx = (cidx0, cidx1)
    cdidx = (cdidx0, cdidx1)
    cvals = (cvals0, cvals1)
    scidx = (scidx0, scidx1)
    sem_g = (sem_g0, sem_g1)
    sem_s = (sem_s0, sem_s1)
    sem_i = (sem_i0, sem_i1)

    with jax.named_scope("acc_zero"):
        def _zrow(i, _):
            for j in range(D // 16):
                zbuf[i, pl.ds(j * 16, 16)] = jnp.zeros((16,), jnp.float32)
            return 0
        lax.fori_loop(0, ZROWS, _zrow, 0)
        for r in range(ROWS_PER_SUB // ZROWS):
            pltpu.sync_copy(zbuf, acc.at[pl.ds(s * ROWS_PER_SUB + r * ZROWS, ZROWS)])
        plsc.subcore_barrier()

    def _prefetch_idx(t, b):
        pltpu.async_copy(sidx.at[pl.ds(t * CHUNK, CHUNK)], cidx[b], sem_i[b])
        pltpu.async_copy(didx.at[pl.ds(t * CHUNK, CHUNK)], cdidx[b], sem_i[b])
        pltpu.async_copy(vals.at[pl.ds(t * CHUNK, CHUNK)], cvals[b], sem_i[b])

    def _idx_wait(b):
        pltpu.make_async_copy(sidx.at[pl.ds(0, CHUNK)], cidx[b], sem_i[b]).wait()
        pltpu.make_async_copy(didx.at[pl.ds(0, CHUNK)], cdidx[b], sem_i[b]).wait()
        pltpu.make_async_copy(vals.at[pl.ds(0, CHUNK)], cvals[b], sem_i[b]).wait()

    def _gather_issue(b):
        pltpu.async_copy(table.at[cidx[b]], rows[b], sem_g[b])

    def _gather_wait(b):
        pltpu.make_async_copy(table.at[pl.ds(0, CHUNK)], rows[b], sem_g[b]).wait()

    def _scatter_issue(b):
        for j in range(CHUNK // 16):
            sl = pl.ds(j * 16, 16)
            scidx[b][sl] = cdidx[b][sl]
        pltpu.async_copy(rows[b], acc.at[scidx[b]], sem_s[b], add=True)

    def _scatter_wait(b):
        pltpu.make_async_copy(table.at[pl.ds(0, CHUNK)], rows[b], sem_s[b]).wait()

    def _scale(b):
        def _g(g, _):
            vv = cvals[b][pl.ds(g * 16, 16)]
            for k in range(16):
                sv = lax.broadcast(vv[k], (16,))
                row = g * 16 + k
                for j in range(D // 16):
                    sl = pl.ds(j * 16, 16)
                    rows[b][row, sl] = rows[b][row, sl] * sv
            return 0
        lax.fori_loop(0, CHUNK // 16, _g, 0)

    def _chunk(t, b, wait_prev_scatter, issue_next, prefetch2):
        _gather_wait(b)
        if wait_prev_scatter:
            _scatter_wait(1 - b)
        if issue_next:
            _idx_wait(1 - b)
            _gather_issue(1 - b)
        _scale(b)
        _scatter_issue(b)
        if prefetch2:
            _prefetch_idx(t + 2, b)

    def _pipeline(bc, n):
        pltpu.sync_copy(sidx.at[pl.ds(bc * CHUNK, CHUNK)], cidx[0])
        pltpu.sync_copy(didx.at[pl.ds(bc * CHUNK, CHUNK)], cdidx[0])
        pltpu.sync_copy(vals.at[pl.ds(bc * CHUNK, CHUNK)], cvals[0])
        _gather_issue(0)
        _prefetch_idx(bc + 1, 1)
        _chunk(bc, 0, False, True, True)
        _chunk(bc + 1, 1, True, True, True)

        def _pair(i, _):
            _chunk(bc + 2 * i, 0, True, True, True)
            _chunk(bc + 2 * i + 1, 1, True, True, True)
            return 0
        lax.fori_loop(1, n // 2 - 1, _pair, 0)

        _chunk(bc + n - 2, 0, True, True, False)
        _chunk(bc + n - 1, 1, True, False, False)
        _scatter_wait(1)

    with jax.named_scope("edge_pipe"):
        @pl.when(c == 0)
        def _core0():
            _pipeline(s * CPW0, CPW0)

        @pl.when(c == 1)
        def _core1():
            _pipeline(NS * CPW0 + s * CPW1, CPW1)

        plsc.subcore_barrier()

    with jax.named_scope("acc_flush"):
        pltpu.sync_copy(acc.at[pl.ds(s * ROWS_PER_SUB, ROWS_PER_SUB)],
                        out.at[c, pl.ds(s * ROWS_PER_SUB, ROWS_PER_SUB)])


@jax.jit
def _segsum(table, sidx, didx, vals):
    mesh = plsc.VectorSubcoreMesh(core_axis_name="c", subcore_axis_name="s")
    f = functools.partial(
        pl.kernel,
        mesh=mesh,
        out_type=jax.ShapeDtypeStruct((NC, NP, D), jnp.float32),
        scratch_types=[
            pltpu.VMEM((ZROWS, D), jnp.float32),
            pltpu.VMEM((CHUNK, D), jnp.float32),
            pltpu.VMEM((CHUNK, D), jnp.float32),
            pltpu.VMEM((CHUNK,), jnp.int32),
            pltpu.VMEM((CHUNK,), jnp.int32),
            pltpu.VMEM((CHUNK,), jnp.int32),
            pltpu.VMEM((CHUNK,), jnp.int32),
            pltpu.VMEM((CHUNK,), jnp.float32),
            pltpu.VMEM((CHUNK,), jnp.float32),
            pltpu.VMEM((CHUNK,), jnp.int32),
            pltpu.VMEM((CHUNK,), jnp.int32),
            pltpu.SemaphoreType.DMA,
            pltpu.SemaphoreType.DMA,
            pltpu.SemaphoreType.DMA,
            pltpu.SemaphoreType.DMA,
            pltpu.SemaphoreType.DMA,
            pltpu.SemaphoreType.DMA,
            pltpu.VMEM_SHARED((NP, D), jnp.float32),
        ],
    )(_segsum_kernel)
    return f(table, sidx, didx, vals)[:, :N]


def _ho_body(p_ref, w_ref, b_ref, o_ref):
    x = p_ref[0] + p_ref[1]
    y = jnp.dot(x, w_ref[...], preferred_element_type=jnp.float32) + b_ref[...]
    o_ref[...] = jnp.where(y >= 0, y, ALPHA * y)


@jax.jit
def _ho(partials, w, b):
    blk = 1000
    grid = N // blk
    return pl.pallas_call(
        _ho_body,
        grid=(grid,),
        in_specs=[
            pl.BlockSpec((NC, blk, D), lambda i: (0, i, 0)),
            pl.BlockSpec((D, D), lambda i: (0, 0)),
            pl.BlockSpec((1, D), lambda i: (0, 0)),
        ],
        out_specs=pl.BlockSpec((blk, D), lambda i: (i, 0)),
        out_shape=jax.ShapeDtypeStruct((N, D), jnp.float32),
    )(partials, w, b.reshape(1, D))


def _final_body(p_ref, fea_ref, wt_ref, wb_ref, b_ref, o_ref):
    x = p_ref[0] + p_ref[1]
    x = jnp.where(x >= 0, x, ALPHA * x)
    y = (jnp.dot(x, wt_ref[...], preferred_element_type=jnp.float32)
         + jnp.dot(fea_ref[...], wb_ref[...], preferred_element_type=jnp.float32)
         + b_ref[...])
    o_ref[...] = jnp.maximum(y, 0.0)


@jax.jit
def _final(partials, fea, w_top, w_bot, b):
    blk = 1000
    grid = N // blk
    return pl.pallas_call(
        _final_body,
        grid=(grid,),
        in_specs=[
            pl.BlockSpec((NC, blk, D), lambda i: (0, i, 0)),
            pl.BlockSpec((blk, D), lambda i: (i, 0)),
            pl.BlockSpec((D, D), lambda i: (0, 0)),
            pl.BlockSpec((D, D), lambda i: (0, 0)),
            pl.BlockSpec((1, D), lambda i: (0, 0)),
        ],
        out_specs=pl.BlockSpec((blk, D), lambda i: (i, 0)),
        out_shape=jax.ShapeDtypeStruct((N, D), jnp.float32),
    )(partials, fea, w_top, w_bot, b.reshape(1, D))


def kernel(ufea, vfea, edge_index, uv_vals, vu_vals, gc1_W, gc1_b, gc2_W,
           gc2_b, user_union_W, user_union_b, item_union_W, item_union_b):
    u_idx = edge_index[0].astype(jnp.int32)
    v_idx = edge_index[1].astype(jnp.int32)
    pad = E_PAD - E
    u_pad = jnp.concatenate([u_idx, jnp.zeros((pad,), jnp.int32)])
    v_pad = jnp.concatenate([v_idx, jnp.zeros((pad,), jnp.int32)])
    uv_pad = jnp.concatenate([uv_vals, jnp.zeros((pad,), jnp.float32)])
    vu_pad = jnp.concatenate([vu_vals, jnp.zeros((pad,), jnp.float32)])

    s1 = _segsum(ufea, u_pad, v_pad, vu_pad)
    s2 = _segsum(vfea, v_pad, u_pad, uv_pad)
    user_ho = _ho(s1, gc1_W, gc1_b)
    item_ho = _ho(s2, gc2_W, gc2_b)

    s3 = _segsum(user_ho, v_pad, u_pad, uv_pad)
    s4 = _segsum(item_ho, u_pad, v_pad, vu_pad)

    user = _final(s3, ufea, user_union_W[:D], user_union_W[D:], user_union_b)
    item = _final(s4, vfea, item_union_W[:D], item_union_W[D:], item_union_b)
    return (user, item)


# speedup vs baseline: 1.2428x; 1.1411x over previous
import functools

import jax
import jax.numpy as jnp
from jax import lax
from jax.experimental import pallas as pl
from jax.experimental.pallas import tpu as pltpu
from jax.experimental.pallas import tpu_sc as plsc

N = 10000
D = 128
E = 320000
ALPHA = 0.1

NC = 2
NS = 16
CHUNK = 128
CPW0 = 158
E_PAD = NS * CHUNK * CPW0
NP = 10240
ROWS_PER_SUB = NP // NS
ZROWS = 64


def _segsum_kernel(table, sidx, didx, vals, out,
                   zbuf, rows0, rows1,
                   cidx0, cidx1, cdidx0, cdidx1, cvals0, cvals1,
                   scidx0, scidx1,
                   sem_g0, sem_g1, sem_s0, sem_s1, sem_i0, sem_i1, acc):
    c = lax.axis_index("c")
    s = lax.axis_index("s")
    rows = (rows0, rows1)
    cidx = (cidx0, cidx1)
    cdidx = (cdidx0, cdidx1)
    cvals = (cvals0, cvals1)
    scidx = (scidx0, scidx1)
    sem_g = (sem_g0, sem_g1)
    sem_s = (sem_s0, sem_s1)
    sem_i = (sem_i0, sem_i1)

    with jax.named_scope("acc_zero"):
        @pl.when(c == 0)
        def _zero():
            def _zrow(i, _):
                for j in range(D // 16):
                    zbuf[i, pl.ds(j * 16, 16)] = jnp.zeros((16,), jnp.float32)
                return 0
            lax.fori_loop(0, ZROWS, _zrow, 0)
            for r in range(ROWS_PER_SUB // ZROWS):
                pltpu.sync_copy(zbuf, acc.at[pl.ds(s * ROWS_PER_SUB + r * ZROWS, ZROWS)])
        plsc.subcore_barrier()

    def _prefetch_idx(t, b):
        pltpu.async_copy(sidx.at[pl.ds(t * CHUNK, CHUNK)], cidx[b], sem_i[b])
        pltpu.async_copy(didx.at[pl.ds(t * CHUNK, CHUNK)], cdidx[b], sem_i[b])
        pltpu.async_copy(vals.at[pl.ds(t * CHUNK, CHUNK)], cvals[b], sem_i[b])

    def _idx_wait(b):
        pltpu.make_async_copy(sidx.at[pl.ds(0, CHUNK)], cidx[b], sem_i[b]).wait()
        pltpu.make_async_copy(didx.at[pl.ds(0, CHUNK)], cdidx[b], sem_i[b]).wait()
        pltpu.make_async_copy(vals.at[pl.ds(0, CHUNK)], cvals[b], sem_i[b]).wait()

    def _gather_issue(b):
        pltpu.async_copy(table.at[cidx[b]], rows[b], sem_g[b])

    def _gather_wait(b):
        pltpu.make_async_copy(table.at[pl.ds(0, CHUNK)], rows[b], sem_g[b]).wait()

    def _scatter_issue(b):
        for j in range(CHUNK // 16):
            sl = pl.ds(j * 16, 16)
            scidx[b][sl] = cdidx[b][sl]
        pltpu.async_copy(rows[b], acc.at[scidx[b]], sem_s[b], add=True)

    def _scatter_wait(b):
        pltpu.make_async_copy(table.at[pl.ds(0, CHUNK)], rows[b], sem_s[b]).wait()

    def _scale(b):
        def _g(g, _):
            vv = cvals[b][pl.ds(g * 16, 16)]
            for k in range(16):
                sv = lax.broadcast(vv[k], (16,))
                row = g * 16 + k
                for j in range(D // 16):
                    sl = pl.ds(j * 16, 16)
                    rows[b][row, sl] = rows[b][row, sl] * sv
            return 0
        lax.fori_loop(0, CHUNK // 16, _g, 0)

    def _chunk(t, b, wait_prev_scatter, issue_next, prefetch2):
        _gather_wait(b)
        if wait_prev_scatter:
            _scatter_wait(1 - b)
        if issue_next:
            _idx_wait(1 - b)
            _gather_issue(1 - b)
        _scale(b)
        _scatter_issue(b)
        if prefetch2:
            _prefetch_idx(t + 2, b)

    def _pipeline(bc, n):
        pltpu.sync_copy(sidx.at[pl.ds(bc * CHUNK, CHUNK)], cidx[0])
        pltpu.sync_copy(didx.at[pl.ds(bc * CHUNK, CHUNK)], cdidx[0])
        pltpu.sync_copy(vals.at[pl.ds(bc * CHUNK, CHUNK)], cvals[0])
        _gather_issue(0)
        _prefetch_idx(bc + 1, 1)
        _chunk(bc, 0, False, True, True)
        _chunk(bc + 1, 1, True, True, True)

        def _pair(i, _):
            _chunk(bc + 2 * i, 0, True, True, True)
            _chunk(bc + 2 * i + 1, 1, True, True, True)
            return 0
        lax.fori_loop(1, n // 2 - 1, _pair, 0)

        _chunk(bc + n - 2, 0, True, True, False)
        _chunk(bc + n - 1, 1, True, False, False)
        _scatter_wait(1)

    with jax.named_scope("edge_pipe"):
        @pl.when(c == 0)
        def _core0():
            _pipeline(s * CPW0, CPW0)

        plsc.subcore_barrier()

    with jax.named_scope("acc_flush"):
        @pl.when(c == 0)
        def _flush():
            pltpu.sync_copy(acc.at[pl.ds(s * ROWS_PER_SUB, ROWS_PER_SUB)],
                            out.at[pl.ds(s * ROWS_PER_SUB, ROWS_PER_SUB)])


@jax.jit
def _segsum(table, sidx, didx, vals):
    mesh = plsc.VectorSubcoreMesh(core_axis_name="c", subcore_axis_name="s")
    f = functools.partial(
        pl.kernel,
        mesh=mesh,
        out_type=jax.ShapeDtypeStruct((NP, D), jnp.float32),
        scratch_types=[
            pltpu.VMEM((ZROWS, D), jnp.float32),
            pltpu.VMEM((CHUNK, D), jnp.float32),
            pltpu.VMEM((CHUNK, D), jnp.float32),
            pltpu.VMEM((CHUNK,), jnp.int32),
            pltpu.VMEM((CHUNK,), jnp.int32),
            pltpu.VMEM((CHUNK,), jnp.int32),
            pltpu.VMEM((CHUNK,), jnp.int32),
            pltpu.VMEM((CHUNK,), jnp.float32),
            pltpu.VMEM((CHUNK,), jnp.float32),
            pltpu.VMEM((CHUNK,), jnp.int32),
            pltpu.VMEM((CHUNK,), jnp.int32),
            pltpu.SemaphoreType.DMA,
            pltpu.SemaphoreType.DMA,
            pltpu.SemaphoreType.DMA,
            pltpu.SemaphoreType.DMA,
            pltpu.SemaphoreType.DMA,
            pltpu.SemaphoreType.DMA,
            pltpu.VMEM_SHARED((NP, D), jnp.float32),
        ],
    )(_segsum_kernel)
    return f(table, sidx, didx, vals)[:N]


def _ho_body(p_ref, w_ref, b_ref, o_ref):
    x = p_ref[...]
    y = jnp.dot(x, w_ref[...], preferred_element_type=jnp.float32) + b_ref[...]
    o_ref[...] = jnp.where(y >= 0, y, ALPHA * y)


@jax.jit
def _ho(ssum, w, b):
    blk = 1000
    grid = N // blk
    return pl.pallas_call(
        _ho_body,
        grid=(grid,),
        in_specs=[
            pl.BlockSpec((blk, D), lambda i: (i, 0)),
            pl.BlockSpec((D, D), lambda i: (0, 0)),
            pl.BlockSpec((1, D), lambda i: (0, 0)),
        ],
        out_specs=pl.BlockSpec((blk, D), lambda i: (i, 0)),
        out_shape=jax.ShapeDtypeStruct((N, D), jnp.float32),
    )(ssum, w, b.reshape(1, D))


def _final_body(p_ref, fea_ref, wt_ref, wb_ref, b_ref, o_ref):
    x = p_ref[...]
    x = jnp.where(x >= 0, x, ALPHA * x)
    y = (jnp.dot(x, wt_ref[...], preferred_element_type=jnp.float32)
         + jnp.dot(fea_ref[...], wb_ref[...], preferred_element_type=jnp.float32)
         + b_ref[...])
    o_ref[...] = jnp.maximum(y, 0.0)


@jax.jit
def _final(ssum, fea, w_top, w_bot, b):
    blk = 1000
    grid = N // blk
    return pl.pallas_call(
        _final_body,
        grid=(grid,),
        in_specs=[
            pl.BlockSpec((blk, D), lambda i: (i, 0)),
            pl.BlockSpec((blk, D), lambda i: (i, 0)),
            pl.BlockSpec((D, D), lambda i: (0, 0)),
            pl.BlockSpec((D, D), lambda i: (0, 0)),
            pl.BlockSpec((1, D), lambda i: (0, 0)),
        ],
        out_specs=pl.BlockSpec((blk, D), lambda i: (i, 0)),
        out_shape=jax.ShapeDtypeStruct((N, D), jnp.float32),
    )(ssum, fea, w_top, w_bot, b.reshape(1, D))


def kernel(ufea, vfea, edge_index, uv_vals, vu_vals, gc1_W, gc1_b, gc2_W,
           gc2_b, user_union_W, user_union_b, item_union_W, item_union_b):
    u_idx = edge_index[0].astype(jnp.int32)
    v_idx = edge_index[1].astype(jnp.int32)
    pad = E_PAD - E
    u_pad = jnp.concatenate([u_idx, jnp.zeros((pad,), jnp.int32)])
    v_pad = jnp.concatenate([v_idx, jnp.zeros((pad,), jnp.int32)])
    uv_pad = jnp.concatenate([uv_vals, jnp.zeros((pad,), jnp.float32)])
    vu_pad = jnp.concatenate([vu_vals, jnp.zeros((pad,), jnp.float32)])

    s1 = _segsum(ufea, u_pad, v_pad, vu_pad)
    s2 = _segsum(vfea, v_pad, u_pad, uv_pad)
    user_ho = _ho(s1, gc1_W, gc1_b)
    item_ho = _ho(s2, gc2_W, gc2_b)

    s3 = _segsum(user_ho, v_pad, u_pad, uv_pad)
    s4 = _segsum(item_ho, u_pad, v_pad, vu_pad)

    user = _final(s3, ufea, user_union_W[:D], user_union_W[D:], user_union_b)
    item = _final(s4, vfea, item_union_W[:D], item_union_W[D:], item_union_b)
    return (user, item)
